# baseline (speedup 1.0000x reference)
import functools
import math

import jax
import jax.numpy as jnp
from jax import lax
from jax.experimental import pallas as pl
from jax.experimental.pallas import tpu as pltpu
from jax.experimental.pallas import tpu_sc as plsc

D = 64
SCALE = math.sqrt(D)
LANES = 16

NC, NS = 2, 16
NW = NC * NS
B = 4096 * 200
N_PER_W = B // NW
C = 512
N_CHUNKS = N_PER_W // C

_mesh = plsc.VectorSubcoreMesh(core_axis_name="c", subcore_axis_name="s")


@functools.partial(
    pl.kernel,
    mesh=_mesh,
    compiler_params=pltpu.CompilerParams(use_tc_tiling_on_sc=False),
    out_type=jax.ShapeDtypeStruct((B, D), jnp.float32),
    scratch_types=[
        pltpu.VMEM((C,), jnp.int32),
        pltpu.VMEM((C, D), jnp.float32),
        pltpu.SemaphoreType.DMA,
    ],
)
def _embed(tokens_hbm, table_hbm, out_hbm, idx_v, rows_v, sem):
    wid = lax.axis_index("s") * NC + lax.axis_index("c")
    base = wid * N_PER_W

    def chunk(ci, carry):
        off = base + ci * C
        pltpu.sync_copy(tokens_hbm.at[pl.ds(off, C)], idx_v)
        pltpu.async_copy(table_hbm.at[idx_v], rows_v, sem).wait()

        def row(i, c2):
            for j in range(D // LANES):
                s = pl.ds(j * LANES, LANES)
                rows_v[i, s] = rows_v[i, s] * SCALE
            return c2

        lax.fori_loop(0, C, row, 0)
        pltpu.sync_copy(rows_v, out_hbm.at[pl.ds(off, C)])
        return carry

    lax.fori_loop(0, N_CHUNKS, chunk, 0)


def kernel(tokens, table):
    idx = tokens.reshape(-1).astype(jnp.int32)
    out = _embed(idx, table)
    return out.reshape(tokens.shape + (D,))
